# baseline (speedup 1.0000x reference)
import functools

import jax
import jax.numpy as jnp
from jax import lax
from jax.experimental import pallas as pl
from jax.experimental.pallas import tpu as pltpu
from jax.experimental.pallas import tpu_sc as plsc

VOCAB = 1000000
DIM = 64
B = 16384
L = 50
EPS = 1e-12

NC = 2
NS = 16
NW = NC * NS
TOTAL = B * L
PER_W = TOTAL // NW
CHUNK = 128
NCHUNK = PER_W // CHUNK
NBUF = 4
SEG = DIM // 16


def _rsqrt(x):
    i = plsc.bitcast(x, jnp.int32)
    i = jnp.full((16,), 0x5F3759DF, jnp.int32) - lax.shift_right_arithmetic(
        i, jnp.full((16,), 1, jnp.int32))
    y = plsc.bitcast(i, jnp.float32)
    half = x * 0.5
    for _ in range(3):
        y = y * (1.5 - half * y * y)
    return y


def _ln_chunk(buf, g_v, b_v):
    g = [g_v[pl.ds(k * 16, 16)] for k in range(SEG)]
    b = [b_v[pl.ds(k * 16, 16)] for k in range(SEG)]

    def row(i, carry):
        r = buf.at[i]
        v = [r[pl.ds(k * 16, 16)] for k in range(SEG)]
        s = (v[0] + v[1]) + (v[2] + v[3])
        q = (v[0] * v[0] + v[1] * v[1]) + (v[2] * v[2] + v[3] * v[3])
        ssum = jnp.sum(s)
        qsum = jnp.sum(q)
        mean = jnp.full((16,), ssum, jnp.float32) * (1.0 / DIM)
        var = (jnp.full((16,), qsum, jnp.float32) * (1.0 / DIM)
               - mean * mean + EPS)
        rstd = _rsqrt(var)
        for k in range(SEG):
            r[pl.ds(k * 16, 16)] = (v[k] - mean) * rstd * g[k] + b[k]
        return carry

    lax.fori_loop(0, CHUNK, row, 0)


def _body(ids_hbm, table_hbm, g_hbm, b_hbm, out_hbm,
          idx_v, rows_v, g_v, b_v, gsems, osems):
    cid = lax.axis_index("c")
    sid = lax.axis_index("s")
    wid = sid * NC + cid
    base = wid * PER_W

    pltpu.sync_copy(ids_hbm.at[wid], idx_v)
    pltpu.sync_copy(g_hbm, g_v)
    pltpu.sync_copy(b_hbm, b_v)

    def gather_desc(c, r):
        return pltpu.make_async_copy(
            table_hbm.at[idx_v.at[c]], rows_v.at[r], gsems[r])

    def out_desc(c, r):
        return pltpu.make_async_copy(
            rows_v.at[r], out_hbm.at[pl.ds(base + c * CHUNK, CHUNK)],
            osems[r])

    gather_desc(0, 0).start()
    gather_desc(1, 1).start()

    def outer(c4, carry):
        for k in range(NBUF):
            c = c4 * NBUF + k
            r = k
            rn = (k + 2) % NBUF

            def prefetch():
                @pl.when(c >= 2)
                def _():
                    out_desc(c - 2, rn).wait()
                gather_desc(c + 2, rn).start()

            if k < 2:
                prefetch()
            else:
                @pl.when(c4 <= NCHUNK // NBUF - 2)
                def _():
                    prefetch()

            gather_desc(c, r).wait()
            _ln_chunk(rows_v.at[r], g_v, b_v)
            out_desc(c, r).start()
        return carry

    lax.fori_loop(0, NCHUNK // NBUF, outer, 0)

    for k in range(NBUF):
        c = NCHUNK - NBUF + k
        out_desc(c, k % NBUF).wait()


def kernel(input_ids, table, gamma, beta):
    ids3 = input_ids.reshape(NW, NCHUNK, CHUNK).astype(jnp.int32)

    mesh = plsc.VectorSubcoreMesh(core_axis_name="c", subcore_axis_name="s")
    run = pl.kernel(
        _body,
        out_type=jax.ShapeDtypeStruct((TOTAL, DIM), jnp.float32),
        mesh=mesh,
        compiler_params=pltpu.CompilerParams(
            needs_layout_passes=False, use_tc_tiling_on_sc=False),
        scratch_types=[
            pltpu.VMEM((NCHUNK, CHUNK), jnp.int32),
            pltpu.VMEM((NBUF, CHUNK, DIM), jnp.float32),
            pltpu.VMEM((DIM,), jnp.float32),
            pltpu.VMEM((DIM,), jnp.float32),
            [pltpu.SemaphoreType.DMA] * NBUF,
            [pltpu.SemaphoreType.DMA] * NBUF,
        ],
    )
    out = run(ids3, table, gamma, beta)
    return out.reshape(B, L, DIM)


# speedup vs baseline: 1.5458x; 1.5458x over previous
import functools

import jax
import jax.numpy as jnp
from jax import lax
from jax.experimental import pallas as pl
from jax.experimental.pallas import tpu as pltpu
from jax.experimental.pallas import tpu_sc as plsc

VOCAB = 1000000
DIM = 64
B = 16384
L = 50
EPS = 1e-12

NC = 2
NS = 16
NW = NC * NS
TOTAL = B * L
PER_W = TOTAL // NW
CHUNK = 128
NCHUNK = PER_W // CHUNK
NBUF = 4
SEG = DIM // 16


def _rsqrt(x):
    i = plsc.bitcast(x, jnp.int32)
    i = jnp.full((16,), 0x5F3759DF, jnp.int32) - lax.shift_right_arithmetic(
        i, jnp.full((16,), 1, jnp.int32))
    y = plsc.bitcast(i, jnp.float32)
    half = x * 0.5
    for _ in range(3):
        y = y * (1.5 - half * y * y)
    return y


def _ln_chunk(buf, g_v, b_v):
    g = [g_v[pl.ds(k * 16, 16)] for k in range(SEG)]
    b = [b_v[pl.ds(k * 16, 16)] for k in range(SEG)]

    @plsc.parallel_loop(0, CHUNK, 1, unroll=4)
    def row(i):
        r = buf.at[i]
        v = [r[pl.ds(k * 16, 16)] for k in range(SEG)]
        s = (v[0] + v[1]) + (v[2] + v[3])
        q = (v[0] * v[0] + v[1] * v[1]) + (v[2] * v[2] + v[3] * v[3])
        ssum = jnp.sum(s)
        qsum = jnp.sum(q)
        mean = jnp.full((16,), ssum, jnp.float32) * (1.0 / DIM)
        var = (jnp.full((16,), qsum, jnp.float32) * (1.0 / DIM)
               - mean * mean + EPS)
        rstd = _rsqrt(var)
        for k in range(SEG):
            r[pl.ds(k * 16, 16)] = (v[k] - mean) * rstd * g[k] + b[k]


def _body(ids_hbm, table_hbm, g_hbm, b_hbm, out_hbm,
          idx_v, rows_v, g_v, b_v, gsems, osems):
    cid = lax.axis_index("c")
    sid = lax.axis_index("s")
    wid = sid * NC + cid
    base = wid * PER_W

    pltpu.sync_copy(ids_hbm.at[wid], idx_v)
    pltpu.sync_copy(g_hbm, g_v)
    pltpu.sync_copy(b_hbm, b_v)

    def gather_desc(c, r):
        return pltpu.make_async_copy(
            table_hbm.at[idx_v.at[c]], rows_v.at[r], gsems[r])

    def out_desc(c, r):
        return pltpu.make_async_copy(
            rows_v.at[r], out_hbm.at[pl.ds(base + c * CHUNK, CHUNK)],
            osems[r])

    gather_desc(0, 0).start()
    gather_desc(1, 1).start()

    def outer(c4, carry):
        for k in range(NBUF):
            c = c4 * NBUF + k
            r = k
            rn = (k + 2) % NBUF

            def prefetch():
                @pl.when(c >= 2)
                def _():
                    out_desc(c - 2, rn).wait()
                gather_desc(c + 2, rn).start()

            if k < 2:
                prefetch()
            else:
                @pl.when(c4 <= NCHUNK // NBUF - 2)
                def _():
                    prefetch()

            gather_desc(c, r).wait()
            _ln_chunk(rows_v.at[r], g_v, b_v)
            out_desc(c, r).start()
        return carry

    lax.fori_loop(0, NCHUNK // NBUF, outer, 0)

    for k in range(NBUF):
        c = NCHUNK - NBUF + k
        out_desc(c, k % NBUF).wait()


def kernel(input_ids, table, gamma, beta):
    ids3 = input_ids.reshape(NW, NCHUNK, CHUNK).astype(jnp.int32)

    mesh = plsc.VectorSubcoreMesh(core_axis_name="c", subcore_axis_name="s")
    run = pl.kernel(
        _body,
        out_type=jax.ShapeDtypeStruct((TOTAL, DIM), jnp.float32),
        mesh=mesh,
        compiler_params=pltpu.CompilerParams(
            needs_layout_passes=False, use_tc_tiling_on_sc=False),
        scratch_types=[
            pltpu.VMEM((NCHUNK, CHUNK), jnp.int32),
            pltpu.VMEM((NBUF, CHUNK, DIM), jnp.float32),
            pltpu.VMEM((DIM,), jnp.float32),
            pltpu.VMEM((DIM,), jnp.float32),
            [pltpu.SemaphoreType.DMA] * NBUF,
            [pltpu.SemaphoreType.DMA] * NBUF,
        ],
    )
    out = run(ids3, table, gamma, beta)
    return out.reshape(B, L, DIM)


# speedup vs baseline: 1.5488x; 1.0019x over previous
import functools

import jax
import jax.numpy as jnp
from jax import lax
from jax.experimental import pallas as pl
from jax.experimental.pallas import tpu as pltpu
from jax.experimental.pallas import tpu_sc as plsc

VOCAB = 1000000
DIM = 64
B = 16384
L = 50
EPS = 1e-12

NC = 2
NS = 16
NW = NC * NS
TOTAL = B * L
PER_W = TOTAL // NW
CHUNK = 128
NCHUNK = PER_W // CHUNK
NBUF = 8
DEPTH = NBUF // 2
SEG = DIM // 16


def _rsqrt(x):
    i = plsc.bitcast(x, jnp.int32)
    i = jnp.full((16,), 0x5F3759DF, jnp.int32) - lax.shift_right_arithmetic(
        i, jnp.full((16,), 1, jnp.int32))
    y = plsc.bitcast(i, jnp.float32)
    half = x * 0.5
    for _ in range(3):
        y = y * (1.5 - half * y * y)
    return y


def _ln_chunk(buf, g_v, b_v):
    g = [g_v[pl.ds(k * 16, 16)] for k in range(SEG)]
    b = [b_v[pl.ds(k * 16, 16)] for k in range(SEG)]

    @plsc.parallel_loop(0, CHUNK, 1, unroll=4)
    def row(i):
        r = buf.at[i]
        v = [r[pl.ds(k * 16, 16)] for k in range(SEG)]
        s = (v[0] + v[1]) + (v[2] + v[3])
        q = (v[0] * v[0] + v[1] * v[1]) + (v[2] * v[2] + v[3] * v[3])
        ssum = jnp.sum(s)
        qsum = jnp.sum(q)
        mean = jnp.full((16,), ssum, jnp.float32) * (1.0 / DIM)
        var = (jnp.full((16,), qsum, jnp.float32) * (1.0 / DIM)
               - mean * mean + EPS)
        rstd = _rsqrt(var)
        for k in range(SEG):
            r[pl.ds(k * 16, 16)] = (v[k] - mean) * rstd * g[k] + b[k]


def _body(ids_hbm, table_hbm, g_hbm, b_hbm, out_hbm,
          idx_v, rows_v, g_v, b_v, gsems, osems):
    cid = lax.axis_index("c")
    sid = lax.axis_index("s")
    wid = sid * NC + cid
    base = wid * PER_W

    pltpu.sync_copy(ids_hbm.at[wid], idx_v)
    pltpu.sync_copy(g_hbm, g_v)
    pltpu.sync_copy(b_hbm, b_v)

    def gather_desc(c, r):
        return pltpu.make_async_copy(
            table_hbm.at[idx_v.at[c]], rows_v.at[r], gsems[r])

    def out_desc(c, r):
        return pltpu.make_async_copy(
            rows_v.at[r], out_hbm.at[pl.ds(base + c * CHUNK, CHUNK)],
            osems[r])

    for d in range(DEPTH):
        gather_desc(d, d).start()

    def outer(c4, carry):
        for k in range(NBUF):
            c = c4 * NBUF + k
            r = k
            rn = (k + DEPTH) % NBUF

            def prefetch():
                @pl.when(c >= DEPTH)
                def _():
                    out_desc(c - DEPTH, rn).wait()
                gather_desc(c + DEPTH, rn).start()

            if k < NBUF - DEPTH:
                prefetch()
            else:
                @pl.when(c4 <= NCHUNK // NBUF - 2)
                def _():
                    prefetch()

            gather_desc(c, r).wait()
            _ln_chunk(rows_v.at[r], g_v, b_v)
            out_desc(c, r).start()
        return carry

    lax.fori_loop(0, NCHUNK // NBUF, outer, 0)

    for k in range(NBUF):
        c = NCHUNK - NBUF + k
        out_desc(c, k % NBUF).wait()


def kernel(input_ids, table, gamma, beta):
    ids3 = input_ids.reshape(NW, NCHUNK, CHUNK).astype(jnp.int32)

    mesh = plsc.VectorSubcoreMesh(core_axis_name="c", subcore_axis_name="s")
    run = pl.kernel(
        _body,
        out_type=jax.ShapeDtypeStruct((TOTAL, DIM), jnp.float32),
        mesh=mesh,
        compiler_params=pltpu.CompilerParams(
            needs_layout_passes=False, use_tc_tiling_on_sc=False),
        scratch_types=[
            pltpu.VMEM((NCHUNK, CHUNK), jnp.int32),
            pltpu.VMEM((NBUF, CHUNK, DIM), jnp.float32),
            pltpu.VMEM((DIM,), jnp.float32),
            pltpu.VMEM((DIM,), jnp.float32),
            [pltpu.SemaphoreType.DMA] * NBUF,
            [pltpu.SemaphoreType.DMA] * NBUF,
        ],
    )
    out = run(ids3, table, gamma, beta)
    return out.reshape(B, L, DIM)
